# baseline (speedup 1.0000x reference)
import functools

import jax
import jax.numpy as jnp
from jax import lax
from jax.experimental import pallas as pl
from jax.experimental.pallas import tpu as pltpu
from jax.experimental.pallas import tpu_sc as plsc

_INTERPRET = False

NUM_A = 20000
NUM_C = 93
NCLS = 80
PAD_A = 20480
TOPK = 200
NB = 8
NEG = -jnp.inf


def _score_body(x_ref, o_ref):
    x = x_ref[...]
    li = lax.broadcasted_iota(jnp.int32, (NUM_A, NUM_C), 1)
    mask = (li >= 1) & (li <= NCLS)
    o_ref[...] = jnp.max(jnp.where(mask, x, NEG), axis=1, keepdims=True)


def _scores(y0):
    return pl.pallas_call(
        _score_body,
        out_shape=jax.ShapeDtypeStruct((NUM_A, 1), jnp.float32),
        interpret=_INTERPRET,
    )(y0)


def _topk_body(s_ref, idx_ref, sm_ref):
    sm_ref[...] = s_ref[...]
    big = jnp.int32(1 << 30)
    fr = lax.broadcasted_iota(jnp.int32, (160, 128), 0)
    fl = lax.broadcasted_iota(jnp.int32, (160, 128), 1)
    li = lax.broadcasted_iota(jnp.int32, (1, 128), 1)
    cm0 = jnp.max(s_ref[...], axis=0, keepdims=True)

    def _bfly_max(v):
        for k in (64, 32, 16, 8, 4, 2, 1):
            v = jnp.maximum(v, pltpu.roll(v, k, 1))
        return v

    def _bfly_min_i(v):
        for k in (64, 32, 16, 8, 4, 2, 1):
            v = jnp.minimum(v, pltpu.roll(v, k, 1))
        return v

    def body(i, cm):
        mv = _bfly_max(cm)
        s = sm_ref[...]
        eqm = s == jnp.broadcast_to(mv, (160, 128))
        rml = jnp.min(jnp.where(eqm, fr, big), axis=0,
                      keepdims=True)
        posv = _bfly_min_i(jnp.where(cm == mv, rml * 128 + li, big))
        idx_ref[i] = posv[0, 0]
        cv = jnp.bitwise_and(posv, 127)
        pos2d = jnp.broadcast_to(posv, (160, 128))
        s = jnp.where((fr * 128 + fl) == pos2d, NEG, s)
        sm_ref[...] = s
        in_col = fl == jnp.broadcast_to(cv, (160, 128))
        col = jnp.max(jnp.where(in_col, s, NEG), axis=0, keepdims=True)
        return jnp.where(li == cv, col, cm)

    lax.fori_loop(0, TOPK, body, cm0)


def _topk(s2d):
    return pl.pallas_call(
        _topk_body,
        out_shape=jax.ShapeDtypeStruct((TOPK,), jnp.int32),
        out_specs=pl.BlockSpec(memory_space=pltpu.SMEM),
        scratch_shapes=[
            pltpu.VMEM((160, 128), jnp.float32),
        ],
        interpret=_INTERPRET,
    )(s2d)


def _decode_rows(g):
    conf = g[:, :, 1:NCLS + 1]
    m = jnp.max(conf, axis=2, keepdims=True)
    li = lax.broadcasted_iota(jnp.int32, conf.shape, 2)
    am = jnp.min(jnp.where(conf == m, li, 127), axis=2,
                 keepdims=True).astype(jnp.float32)
    thr = (m > 0.5).astype(jnp.float32)

    def ch(c):
        return g[:, :, c:c + 1]

    cx = ch(81) * ch(89) * ch(87) + ch(85)
    cy = ch(82) * ch(90) * ch(88) + ch(86)
    w = jnp.exp(ch(83) * ch(91)) * ch(87)
    h = jnp.exp(ch(84) * ch(92)) * ch(88)
    xmin = (cx - 0.5 * w) * 512.0
    ymin = (cy - 0.5 * h) * 512.0
    xmax = (cx + 0.5 * w) * 512.0
    ymax = (cy + 0.5 * h) * 512.0
    return jnp.concatenate([thr, am, m, xmin, ymin, xmax, ymax], axis=2)


def _gather_body(idx_ref, y_ref, o_ref, g_ref, sem):
    def start(j, _):
        a = idx_ref[j]
        pltpu.make_async_copy(
            y_ref.at[:, pl.ds(a, 1), :],
            g_ref.at[:, pl.ds(j, 1), :],
            sem,
        ).start()
        return 0

    lax.fori_loop(0, TOPK, start, 0)

    def wait(j, _):
        pltpu.make_async_copy(
            y_ref.at[:, pl.ds(0, 1), :],
            g_ref.at[:, pl.ds(0, 1), :],
            sem,
        ).wait()
        return 0

    lax.fori_loop(0, TOPK, wait, 0)
    o_ref[...] = _decode_rows(g_ref[...])


def _gather_decode(idx, y_pred):
    return pl.pallas_call(
        _gather_body,
        out_shape=jax.ShapeDtypeStruct((NB, TOPK, 7), jnp.float32),
        in_specs=[
            pl.BlockSpec(memory_space=pltpu.SMEM),
            pl.BlockSpec(memory_space=pltpu.MemorySpace.HBM),
        ],
        scratch_shapes=[
            pltpu.VMEM((NB, TOPK, NUM_C), jnp.float32),
            pltpu.SemaphoreType.DMA,
        ],
        interpret=_INTERPRET,
    )(idx, y_pred)


_SC_TILES = 25
_ROWS_PER_TILE = 64


def _sc_body(fidx_hbm, y2d_hbm, out_hbm, idxbuf, rows_v, out_v, sem):
    wid = lax.axis_index("s") * 2 + lax.axis_index("c")

    @pl.when(wid < _SC_TILES)
    def _():
        base = wid * _ROWS_PER_TILE
        pltpu.sync_copy(fidx_hbm.at[pl.ds(base, _ROWS_PER_TILE)], idxbuf)

        for g in range(4):
            vec = idxbuf[pl.ds(g * 16, 16)]
            for k in range(16):
                j = g * 16 + k
                pltpu.async_copy(
                    y2d_hbm.at[pl.ds(vec[k], 1), :],
                    rows_v.at[pl.ds(j, 1), :], sem).start()

        def drain(j, _):
            pltpu.make_async_copy(
                y2d_hbm.at[pl.ds(0, 1), :], rows_v.at[pl.ds(0, 1), :],
                sem).wait()
            return 0

        lax.fori_loop(0, _ROWS_PER_TILE, drain, 0)
        li = lax.iota(jnp.int32, 16)
        for g in range(4):
            rows16 = li + g * 16

            def chan(c):
                return plsc.load_gather(
                    rows_v, [rows16, jnp.full((16,), c, jnp.int32)])

            m = chan(1)
            am = jnp.zeros((16,), jnp.int32)
            for c in range(2, NCLS + 1):
                v = chan(c)
                upd = v > m
                m = jnp.where(upd, v, m)
                am = jnp.where(upd, jnp.full((16,), c - 1, jnp.int32), am)
            t87 = chan(87)
            t88 = chan(88)
            cx = chan(81) * chan(89) * t87 + chan(85)
            cy = chan(82) * chan(90) * t88 + chan(86)
            w = jnp.exp(chan(83) * chan(91)) * t87
            h = jnp.exp(chan(84) * chan(92)) * t88
            thr = jnp.where(m > 0.5, 1.0, 0.0).astype(jnp.float32)
            cols = [thr, am.astype(jnp.float32), m,
                    (cx - 0.5 * w) * 512.0, (cy - 0.5 * h) * 512.0,
                    (cx + 0.5 * w) * 512.0, (cy + 0.5 * h) * 512.0]
            for ci, vec in enumerate(cols):
                plsc.store_scatter(
                    out_v, [rows16, jnp.full((16,), ci, jnp.int32)], vec)
        pltpu.sync_copy(out_v, out_hbm.at[pl.ds(base, _ROWS_PER_TILE)])


@functools.cache
def _sc_gather_decode_fn():
    return pl.kernel(
        _sc_body,
        out_type=jax.ShapeDtypeStruct((NB * TOPK, 7), jnp.float32),
        mesh=plsc.VectorSubcoreMesh(core_axis_name="c", subcore_axis_name="s"),
        scratch_types=[
            pltpu.VMEM((_ROWS_PER_TILE,), jnp.int32),
            pltpu.VMEM((_ROWS_PER_TILE, NUM_C), jnp.float32),
            pltpu.VMEM((_ROWS_PER_TILE, 7), jnp.float32),
            pltpu.SemaphoreType.DMA,
        ],
        compiler_params=pltpu.CompilerParams(needs_layout_passes=False),
    )


def _sc_gather_decode(fidx, y2d):
    return _sc_gather_decode_fn()(fidx, y2d)


def kernel(y_pred):
    s = _scores(y_pred[0])
    spad = jnp.concatenate(
        [s[:, 0], jnp.full((PAD_A - NUM_A,), NEG, jnp.float32)])
    idx = _topk(spad.reshape(160, 128))
    fidx = ((jnp.arange(NB, dtype=jnp.int32) * NUM_A)[:, None]
            + idx[None, :]).reshape(NB * TOPK)
    y2d = y_pred.reshape(NB * NUM_A, NUM_C)
    out = _sc_gather_decode(fidx, y2d)
    return out.reshape(NB, TOPK, 7)
